# baseline (speedup 1.0000x reference)
import jax
import jax.numpy as jnp
from jax.experimental import pallas as pl
from jax.experimental.pallas import tpu as pltpu

EMB = 768
H = 8
D = 96
E = 8
S = 128
ES = E * S
A = 2
TN = 2048


def _fused(x_ref, wmh_ref, wmg_ref, router_ref, k_ref, v_ref, exp_ref, out_ref):
    xt = x_ref[...]
    ht_all = jnp.dot(xt, wmh_ref[...].T, preferred_element_type=jnp.float32)
    ohs = []
    for h in range(H):
        ht = ht_all[:, h * D:(h + 1) * D]
        logits = jnp.dot(ht, router_ref[h], preferred_element_type=jnp.float32)
        m1 = jnp.max(logits, axis=-1, keepdims=True)
        m2 = jnp.max(jnp.where(logits < m1, logits, -jnp.inf),
                     axis=-1, keepdims=True)
        el = jnp.exp2((logits - m1) * 1.4426950408889634)
        r = 1.0 / jnp.sum(el, axis=-1, keepdims=True)
        gv = jnp.where(logits >= m2, el * r, 0.0).astype(jnp.bfloat16)

        hs = (ht * 1.4426950408889634).astype(jnp.bfloat16)
        scores = jnp.dot(hs, k_ref[h].T,
                         preferred_element_type=jnp.float32)
        gate = jnp.dot(gv, exp_ref[...],
                       preferred_element_type=jnp.float32
                       ).astype(jnp.bfloat16)
        em1 = (jnp.exp2(scores) - 1.0).astype(jnp.bfloat16)
        hidden = em1 * gate
        ohs.append(jnp.dot(hidden, v_ref[h], preferred_element_type=jnp.float32))
    o_cat = jnp.concatenate(ohs, axis=1)
    out_ref[...] = jnp.dot(o_cat, wmg_ref[...].T, preferred_element_type=jnp.float32)


def kernel(x, W_mh, W_mg, router, K, V):
    B, T, emb = x.shape
    N = B * T
    x2 = x.reshape(N, emb)
    out_call = pl.pallas_call(
        _fused,
        grid=(N // TN,),
        in_specs=[
            pl.BlockSpec((TN, EMB), lambda t: (t, 0)),
            pl.BlockSpec((H * D, EMB), lambda t: (0, 0)),
            pl.BlockSpec((EMB, H * D), lambda t: (0, 0)),
            pl.BlockSpec((H, D, E), lambda t: (0, 0, 0)),
            pl.BlockSpec((H, ES, D), lambda t: (0, 0, 0)),
            pl.BlockSpec((H, ES, D), lambda t: (0, 0, 0)),
            pl.BlockSpec((E, ES), lambda t: (0, 0)),
        ],
        out_specs=pl.BlockSpec((TN, EMB), lambda t: (t, 0)),
        out_shape=jax.ShapeDtypeStruct((N, EMB), jnp.float32),
        compiler_params=pltpu.CompilerParams(
            dimension_semantics=("parallel",),
        ),
    )
    expand = jnp.kron(jnp.eye(E, dtype=jnp.bfloat16),
                      jnp.ones((1, S), jnp.bfloat16))
    out = out_call(x2, W_mh, W_mg, router, K.astype(jnp.bfloat16),
                   V.astype(jnp.bfloat16), expand)
    return out.reshape(B, T, emb)


# speedup vs baseline: 1.0424x; 1.0424x over previous
import jax
import jax.numpy as jnp
from jax.experimental import pallas as pl
from jax.experimental.pallas import tpu as pltpu

EMB = 768
H = 8
D = 96
E = 8
S = 128
ES = E * S
A = 2
TN = 1024


def _fused(x_ref, wmh_ref, wmg_ref, router_ref, k_ref, v_ref, exp_ref, out_ref):
    xt = x_ref[...]
    ht_all = jnp.dot(xt, wmh_ref[...].T, preferred_element_type=jnp.float32)
    ohs = []
    for h in range(H):
        ht = ht_all[:, h * D:(h + 1) * D]
        logits = jnp.dot(ht, router_ref[h], preferred_element_type=jnp.float32)
        m1 = jnp.max(logits, axis=-1, keepdims=True)
        m2 = jnp.max(jnp.where(logits < m1, logits, -jnp.inf),
                     axis=-1, keepdims=True)
        el = jnp.exp2((logits - m1) * 1.4426950408889634)
        r = 1.0 / jnp.sum(el, axis=-1, keepdims=True)
        gv = jnp.where(logits >= m2, el * r, 0.0).astype(jnp.bfloat16)

        hs = (ht * 1.4426950408889634).astype(jnp.bfloat16)
        scores = jnp.dot(hs, k_ref[h].T,
                         preferred_element_type=jnp.float32)
        gate = jnp.dot(gv, exp_ref[...],
                       preferred_element_type=jnp.float32
                       ).astype(jnp.bfloat16)
        em1 = (jnp.exp2(scores) - 1.0).astype(jnp.bfloat16)
        hidden = em1 * gate
        ohs.append(jnp.dot(hidden, v_ref[h], preferred_element_type=jnp.float32))
    o_cat = jnp.concatenate(ohs, axis=1)
    out_ref[...] = jnp.dot(o_cat, wmg_ref[...].T, preferred_element_type=jnp.float32)


def kernel(x, W_mh, W_mg, router, K, V):
    B, T, emb = x.shape
    N = B * T
    x2 = x.reshape(N, emb)
    out_call = pl.pallas_call(
        _fused,
        grid=(N // TN,),
        in_specs=[
            pl.BlockSpec((TN, EMB), lambda t: (t, 0)),
            pl.BlockSpec((H * D, EMB), lambda t: (0, 0)),
            pl.BlockSpec((EMB, H * D), lambda t: (0, 0)),
            pl.BlockSpec((H, D, E), lambda t: (0, 0, 0)),
            pl.BlockSpec((H, ES, D), lambda t: (0, 0, 0)),
            pl.BlockSpec((H, ES, D), lambda t: (0, 0, 0)),
            pl.BlockSpec((E, ES), lambda t: (0, 0)),
        ],
        out_specs=pl.BlockSpec((TN, EMB), lambda t: (t, 0)),
        out_shape=jax.ShapeDtypeStruct((N, EMB), jnp.float32),
        compiler_params=pltpu.CompilerParams(
            dimension_semantics=("parallel",),
        ),
    )
    expand = jnp.kron(jnp.eye(E, dtype=jnp.bfloat16),
                      jnp.ones((1, S), jnp.bfloat16))
    out = out_call(x2, W_mh, W_mg, router, K.astype(jnp.bfloat16),
                   V.astype(jnp.bfloat16), expand)
    return out.reshape(B, T, emb)
